# Initial kernel scaffold; baseline (speedup 1.0000x reference)
import jax
import jax.numpy as jnp
from jax.experimental import pallas as pl


def kernel(mean, va, xt, action):
    raise NotImplementedError("write your pallas kernel here")


# speedup vs baseline: 6.8529x; 6.8529x over previous
import jax
import jax.numpy as jnp
from jax import lax
from jax.experimental import pallas as pl
from jax.experimental.pallas import tpu as pltpu
from jax.experimental.pallas import tpu_sc as plsc

H = 16
DU = 512
D = 1024
B = 128
NB = D // B
K = 8
_PREC = lax.Precision.HIGHEST


def _diag_factor_body(a_ref, r_ref, tinv_ref, a_s):
    a_s[...] = a_ref[...]
    r_ref[...] = jnp.zeros_like(r_ref)
    tinv_ref[...] = jnp.zeros_like(tinv_ref)

    iota_r = lax.broadcasted_iota(jnp.int32, (1, B, 1), 1)
    iota_c = lax.broadcasted_iota(jnp.int32, (1, 1, B), 2)
    iota_l = lax.broadcasted_iota(jnp.int32, (1, B), 1)

    def chol_step(j, _):
        a = a_s[...]
        rowsel = iota_r == j
        lanesel = iota_l == j
        row = jnp.sum(jnp.where(rowsel, a, 0.0), axis=1)
        d = jnp.sum(jnp.where(lanesel, row, 0.0), axis=1, keepdims=True)
        l = row * lax.rsqrt(d)
        l = jnp.where(iota_l >= j, l, 0.0)
        colsel = iota_c == j
        r_ref[...] = jnp.where(colsel, l[:, :, None], r_ref[...])
        a_s[...] = a - l[:, :, None] * l[:, None, :]
        return 0

    lax.fori_loop(0, B, chol_step, 0)

    def inv_step(j, _):
        r = r_ref[...]
        x = tinv_ref[...]
        rowsel = iota_r == j
        lanesel = iota_l == j
        rrow = jnp.sum(jnp.where(rowsel, r, 0.0), axis=1)
        rjj = jnp.sum(jnp.where(lanesel, rrow, 0.0), axis=1, keepdims=True)
        s = jnp.sum(rrow[:, :, None] * x, axis=1)
        e = jnp.where(iota_l == j, 1.0, 0.0)
        newrow = (e - s) / rjj
        tinv_ref[...] = jnp.where(rowsel, newrow[:, None, :], x)
        return 0

    lax.fori_loop(0, B, inv_step, 0)


def _trsm_body(t_ref, tinv_ref, p_ref):
    pr = t_ref[0, B:, :]
    ti = tinv_ref[0]
    p_ref[0] = lax.dot_general(
        pr, ti, (((1,), (1,)), ((), ())), precision=_PREC,
        preferred_element_type=jnp.float32)


def _syrk_body(t_ref, pfull_ref, pstrip_ref, out_ref):
    t = t_ref[0, :, B:]
    pr = pstrip_ref[0]
    pf = pfull_ref[0]
    out_ref[0] = t - lax.dot_general(
        pr, pf, (((1,), (1,)), ((), ())), precision=_PREC,
        preferred_element_type=jnp.float32)


def _solve_logits_body(*refs):
    panel_refs = refs[:NB - 1]
    tinv_refs = refs[NB - 1:2 * NB - 1]
    zf_ref, mean_ref, phi_ref, phij_ref = refs[2 * NB - 1:2 * NB + 3]
    out_ref = refs[2 * NB + 3]

    pvec = {}
    for k in range(NB - 1, -1, -1):
        acc = zf_ref[0, :, k * B:(k + 1) * B]
        for j in range(k + 1, NB):
            rjk = panel_refs[k][0, (j - k - 1) * B:(j - k) * B, :]
            acc = acc - lax.dot_general(
                pvec[j], rjk, (((1,), (0,)), ((), ())), precision=_PREC,
                preferred_element_type=jnp.float32)
        pvec[k] = lax.dot_general(
            acc, tinv_refs[k][0], (((1,), (0,)), ((), ())), precision=_PREC,
            preferred_element_type=jnp.float32)

    logit = jnp.sum(phi_ref[0] * mean_ref[0])
    for k in range(NB):
        logit = logit + jnp.sum(phij_ref[0, :, k * B:(k + 1) * B] * pvec[k])
    pclick = 1.0 / (1.0 + jnp.exp(-logit))
    out_ref[...] = jnp.full((1, 1, B), pclick, jnp.float32)


def _sc_topk_body(p_hbm, out_hbm, p_v, o_v):
    cid = lax.axis_index("c")
    sid = lax.axis_index("s")

    @pl.when(jnp.logical_and(cid == 0, sid == 0))
    def _():
        pltpu.sync_copy(p_hbm, p_v)
        p = p_v[...]
        io = lax.iota(jnp.int32, 16)
        rank = jnp.zeros((16,), jnp.int32)
        for j in range(H):
            pj = jnp.sum(jnp.where(io == j, p, 0.0))
            beats = jnp.logical_or(
                pj > p, jnp.logical_and(pj == p, io > j))
            rank = rank + jnp.where(beats, 1, 0)
        o_v[...] = jnp.where(rank < K, 1.0, 0.0).astype(jnp.float32)
        pltpu.sync_copy(o_v, out_hbm)


def _topk_mask_sc(pclick):
    mesh = plsc.VectorSubcoreMesh(core_axis_name="c", subcore_axis_name="s")
    f = pl.kernel(
        _sc_topk_body, mesh=mesh,
        out_type=jax.ShapeDtypeStruct((H,), jnp.float32),
        scratch_types=[pltpu.VMEM((16,), jnp.float32),
                       pltpu.VMEM((16,), jnp.float32)],
        compiler_params=pltpu.CompilerParams(needs_layout_passes=False))
    return f(pclick)


def _factorize(t):
    panels = []
    tinvs = []
    for k in range(NB):
        m = D - B * k
        r_k, tinv_k = pl.pallas_call(
            _diag_factor_body,
            grid=(1,),
            in_specs=[pl.BlockSpec((H, B, B), lambda i: (0, 0, 0))],
            out_specs=[pl.BlockSpec((H, B, B), lambda i: (0, 0, 0)),
                       pl.BlockSpec((H, B, B), lambda i: (0, 0, 0))],
            out_shape=[jax.ShapeDtypeStruct((H, B, B), jnp.float32),
                       jax.ShapeDtypeStruct((H, B, B), jnp.float32)],
            scratch_shapes=[pltpu.VMEM((H, B, B), jnp.float32)],
        )(t)
        tinvs.append(tinv_k)
        if m > B:
            p_k = pl.pallas_call(
                _trsm_body,
                grid=(H,),
                in_specs=[pl.BlockSpec((1, m, B), lambda h: (h, 0, 0)),
                          pl.BlockSpec((1, B, B), lambda h: (h, 0, 0))],
                out_specs=pl.BlockSpec((1, m - B, B), lambda h: (h, 0, 0)),
                out_shape=jax.ShapeDtypeStruct((H, m - B, B), jnp.float32),
            )(t, tinv_k)
            panels.append(p_k)
            nr = (m - B) // B
            t = pl.pallas_call(
                _syrk_body,
                grid=(H, nr),
                in_specs=[
                    pl.BlockSpec((1, B, m), lambda h, r: (h, r + 1, 0)),
                    pl.BlockSpec((1, m - B, B), lambda h, r: (h, 0, 0)),
                    pl.BlockSpec((1, B, B), lambda h, r: (h, r, 0)),
                ],
                out_specs=pl.BlockSpec((1, B, m - B), lambda h, r: (h, r, 0)),
                out_shape=jax.ShapeDtypeStruct((H, m - B, m - B), jnp.float32),
            )(t, p_k, p_k)
    return panels, tinvs


def _solve_and_logits(panels, tinvs, zf, mean, phi, phij):
    in_specs = []
    for k in range(NB - 1):
        mk = D - B * (k + 1)
        in_specs.append(pl.BlockSpec((1, mk, B), lambda h: (h, 0, 0)))
    for _ in range(NB):
        in_specs.append(pl.BlockSpec((1, B, B), lambda h: (h, 0, 0)))
    for _ in range(4):
        in_specs.append(pl.BlockSpec((1, 1, D), lambda h: (h, 0, 0)))
    out = pl.pallas_call(
        _solve_logits_body,
        grid=(H,),
        in_specs=in_specs,
        out_specs=pl.BlockSpec((1, 1, B), lambda h: (h, 0, 0)),
        out_shape=jax.ShapeDtypeStruct((H, 1, B), jnp.float32),
    )(*panels, *tinvs, zf, mean, phi, phij)
    return out[:, 0, 0]


def kernel(mean, va, xt, action):
    mean = mean.astype(jnp.float32)
    va = va.astype(jnp.float32)
    xt = xt.astype(jnp.float32)
    action = action.astype(jnp.float32)

    z = jax.random.normal(jax.random.key(42), (H, D), dtype=jnp.float32)
    zf = z[:, ::-1].reshape(H, 1, D)

    phi = jnp.concatenate(
        [jnp.broadcast_to(xt[None, :], (H, DU)), action], axis=1)
    phij = phi[:, ::-1].reshape(H, 1, D)
    phi3 = phi.reshape(H, 1, D)
    mean3 = mean.reshape(H, 1, D)

    t = va[:, ::-1, ::-1]

    panels, tinvs = _factorize(t)
    pclick = _solve_and_logits(panels, tinvs, zf, mean3, phi3, phij)
    return _topk_mask_sc(pclick)


# speedup vs baseline: 8.4631x; 1.2350x over previous
import jax
import jax.numpy as jnp
from jax import lax
from jax.experimental import pallas as pl
from jax.experimental.pallas import tpu as pltpu
from jax.experimental.pallas import tpu_sc as plsc

H = 16
DU = 512
D = 1024
B = 128
NB = D // B
K = 8
_PREC = lax.Precision.HIGHEST


def _bmm(x, y):
    return lax.dot_general(
        x, y, (((2,), (1,)), ((0,), (0,))), precision=_PREC,
        preferred_element_type=jnp.float32)


def _diag_factor_body(a_ref, r_ref, tinv_ref, a_s):
    a_s[...] = a_ref[...]

    iota_r = lax.broadcasted_iota(jnp.int32, (1, B, 1), 1)
    iota_c = lax.broadcasted_iota(jnp.int32, (1, 1, B), 2)
    iota_l = lax.broadcasted_iota(jnp.int32, (1, B), 1)

    def chol_step(j, _):
        a = a_s[...]
        rowsel = iota_r == j
        lanesel = iota_l == j
        row = jnp.sum(jnp.where(rowsel, a, 0.0), axis=1)
        d = jnp.sum(jnp.where(lanesel, row, 0.0), axis=1, keepdims=True)
        l = row * lax.rsqrt(d)
        l = jnp.where(iota_l >= j, l, 0.0)
        upd = a - l[:, :, None] * l[:, None, :]
        a_s[...] = jnp.where(iota_c == j, l[:, :, None], upd)
        return 0

    lax.fori_loop(0, B, chol_step, 0)

    r = a_s[...]
    r_ref[...] = r
    eye = jnp.where(iota_r == iota_c, 1.0, 0.0)
    diag = jnp.sum(jnp.where(iota_r == iota_c, r, 0.0), axis=2)
    n = r / diag[:, :, None] - eye
    p = eye - n
    m = n
    for _ in range(6):
        m = _bmm(m, m)
        p = _bmm(p, eye + m)
    tinv_ref[...] = p / diag[:, None, :]


def _trsm_body(t_ref, tinv_ref, p_ref):
    pr = t_ref[0, B:, :]
    ti = tinv_ref[0]
    p_ref[0] = lax.dot_general(
        pr, ti, (((1,), (1,)), ((), ())), precision=_PREC,
        preferred_element_type=jnp.float32)


def _syrk_body(t_ref, pfull_ref, pstrip_ref, out_ref):
    t = t_ref[0, :, B:]
    pr = pstrip_ref[0]
    pf = pfull_ref[0]
    out_ref[0] = t - lax.dot_general(
        pr, pf, (((1,), (1,)), ((), ())), precision=_PREC,
        preferred_element_type=jnp.float32)


def _solve_logits_body(*refs):
    panel_refs = refs[:NB - 1]
    tinv_refs = refs[NB - 1:2 * NB - 1]
    zf_ref, mean_ref, phi_ref, phij_ref = refs[2 * NB - 1:2 * NB + 3]
    out_ref = refs[2 * NB + 3]

    pvec = {}
    for k in range(NB - 1, -1, -1):
        acc = zf_ref[0, :, k * B:(k + 1) * B]
        for j in range(k + 1, NB):
            rjk = panel_refs[k][0, (j - k - 1) * B:(j - k) * B, :]
            acc = acc - lax.dot_general(
                pvec[j], rjk, (((1,), (0,)), ((), ())), precision=_PREC,
                preferred_element_type=jnp.float32)
        pvec[k] = lax.dot_general(
            acc, tinv_refs[k][0], (((1,), (0,)), ((), ())), precision=_PREC,
            preferred_element_type=jnp.float32)

    logit = jnp.sum(phi_ref[0] * mean_ref[0])
    for k in range(NB):
        logit = logit + jnp.sum(phij_ref[0, :, k * B:(k + 1) * B] * pvec[k])
    pclick = 1.0 / (1.0 + jnp.exp(-logit))
    out_ref[...] = jnp.full((1, 1, B), pclick, jnp.float32)


def _sc_topk_body(p_hbm, out_hbm, p_v, o_v):
    cid = lax.axis_index("c")
    sid = lax.axis_index("s")

    @pl.when(jnp.logical_and(cid == 0, sid == 0))
    def _():
        pltpu.sync_copy(p_hbm, p_v)
        p = p_v[...]
        io = lax.iota(jnp.int32, 16)
        rank = jnp.zeros((16,), jnp.int32)
        for j in range(H):
            pj = jnp.sum(jnp.where(io == j, p, 0.0))
            beats = jnp.logical_or(
                pj > p, jnp.logical_and(pj == p, io > j))
            rank = rank + jnp.where(beats, 1, 0)
        o_v[...] = jnp.where(rank < K, 1.0, 0.0).astype(jnp.float32)
        pltpu.sync_copy(o_v, out_hbm)


def _topk_mask_sc(pclick):
    mesh = plsc.VectorSubcoreMesh(core_axis_name="c", subcore_axis_name="s")
    f = pl.kernel(
        _sc_topk_body, mesh=mesh,
        out_type=jax.ShapeDtypeStruct((H,), jnp.float32),
        scratch_types=[pltpu.VMEM((16,), jnp.float32),
                       pltpu.VMEM((16,), jnp.float32)],
        compiler_params=pltpu.CompilerParams(needs_layout_passes=False))
    return f(pclick)


def _factorize(t):
    panels = []
    tinvs = []
    for k in range(NB):
        m = D - B * k
        r_k, tinv_k = pl.pallas_call(
            _diag_factor_body,
            grid=(1,),
            in_specs=[pl.BlockSpec((H, B, B), lambda i: (0, 0, 0))],
            out_specs=[pl.BlockSpec((H, B, B), lambda i: (0, 0, 0)),
                       pl.BlockSpec((H, B, B), lambda i: (0, 0, 0))],
            out_shape=[jax.ShapeDtypeStruct((H, B, B), jnp.float32),
                       jax.ShapeDtypeStruct((H, B, B), jnp.float32)],
            scratch_shapes=[pltpu.VMEM((H, B, B), jnp.float32)],
        )(t)
        tinvs.append(tinv_k)
        if m > B:
            p_k = pl.pallas_call(
                _trsm_body,
                grid=(H,),
                in_specs=[pl.BlockSpec((1, m, B), lambda h: (h, 0, 0)),
                          pl.BlockSpec((1, B, B), lambda h: (h, 0, 0))],
                out_specs=pl.BlockSpec((1, m - B, B), lambda h: (h, 0, 0)),
                out_shape=jax.ShapeDtypeStruct((H, m - B, B), jnp.float32),
            )(t, tinv_k)
            panels.append(p_k)
            nr = (m - B) // B
            t = pl.pallas_call(
                _syrk_body,
                grid=(H, nr),
                in_specs=[
                    pl.BlockSpec((1, B, m), lambda h, r: (h, r + 1, 0)),
                    pl.BlockSpec((1, m - B, B), lambda h, r: (h, 0, 0)),
                    pl.BlockSpec((1, B, B), lambda h, r: (h, r, 0)),
                ],
                out_specs=pl.BlockSpec((1, B, m - B), lambda h, r: (h, r, 0)),
                out_shape=jax.ShapeDtypeStruct((H, m - B, m - B), jnp.float32),
            )(t, p_k, p_k)
    return panels, tinvs


def _solve_and_logits(panels, tinvs, zf, mean, phi, phij):
    in_specs = []
    for k in range(NB - 1):
        mk = D - B * (k + 1)
        in_specs.append(pl.BlockSpec((1, mk, B), lambda h: (h, 0, 0)))
    for _ in range(NB):
        in_specs.append(pl.BlockSpec((1, B, B), lambda h: (h, 0, 0)))
    for _ in range(4):
        in_specs.append(pl.BlockSpec((1, 1, D), lambda h: (h, 0, 0)))
    out = pl.pallas_call(
        _solve_logits_body,
        grid=(H,),
        in_specs=in_specs,
        out_specs=pl.BlockSpec((1, 1, B), lambda h: (h, 0, 0)),
        out_shape=jax.ShapeDtypeStruct((H, 1, B), jnp.float32),
    )(*panels, *tinvs, zf, mean, phi, phij)
    return out[:, 0, 0]


def kernel(mean, va, xt, action):
    mean = mean.astype(jnp.float32)
    va = va.astype(jnp.float32)
    xt = xt.astype(jnp.float32)
    action = action.astype(jnp.float32)

    z = jax.random.normal(jax.random.key(42), (H, D), dtype=jnp.float32)
    zf = z[:, ::-1].reshape(H, 1, D)

    phi = jnp.concatenate(
        [jnp.broadcast_to(xt[None, :], (H, DU)), action], axis=1)
    phij = phi[:, ::-1].reshape(H, 1, D)
    phi3 = phi.reshape(H, 1, D)
    mean3 = mean.reshape(H, 1, D)

    t = va[:, ::-1, ::-1]

    panels, tinvs = _factorize(t)
    pclick = _solve_and_logits(panels, tinvs, zf, mean3, phi3, phij)
    return _topk_mask_sc(pclick)


# speedup vs baseline: 9.6150x; 1.1361x over previous
import jax
import jax.numpy as jnp
from jax import lax
from jax.experimental import pallas as pl
from jax.experimental.pallas import tpu as pltpu
from jax.experimental.pallas import tpu_sc as plsc

H = 16
DU = 512
D = 1024
B = 128
NB = D // B
K = 8
_PREC = lax.Precision.HIGHEST


def _bmm(x, y):
    return lax.dot_general(
        x, y, (((2,), (1,)), ((0,), (0,))), precision=_PREC,
        preferred_element_type=jnp.float32)


def _diag_factor_body(a_ref, r_ref, tinv_ref, a_s):
    a_s[...] = a_ref[...]

    iota_r = lax.broadcasted_iota(jnp.int32, (1, B, 1), 1)
    iota_c = lax.broadcasted_iota(jnp.int32, (1, 1, B), 2)

    def chol_step(j, _):
        a = a_s[...]
        rowsel = iota_r == j
        colsel = iota_c == j
        row = jnp.sum(jnp.where(rowsel, a, 0.0), axis=1, keepdims=True)
        col = jnp.sum(jnp.where(colsel, a, 0.0), axis=2, keepdims=True)
        d = jnp.sum(jnp.where(colsel, row, 0.0), axis=2, keepdims=True)
        rs = lax.rsqrt(d)
        lrow = jnp.where(iota_c >= j, row * rs, 0.0)
        lcol = jnp.where(iota_r >= j, col * rs, 0.0)
        upd = a - lcol * lrow
        a_s[...] = jnp.where(colsel, lcol, upd)
        return 0

    lax.fori_loop(0, B, chol_step, 0)

    r = a_s[...]
    r_ref[...] = r
    eye = jnp.where(iota_r == iota_c, 1.0, 0.0)
    diag = jnp.sum(jnp.where(iota_r == iota_c, r, 0.0), axis=2)
    n = r / diag[:, :, None] - eye
    p = eye - n
    m = n
    for _ in range(6):
        m = _bmm(m, m)
        p = _bmm(p, eye + m)
    tinv_ref[...] = p / diag[:, None, :]


def _panel_update_body(t_ref, tinv_ref, p_ref, tout_ref):
    pr = t_ref[0, B:, :B]
    ti = tinv_ref[0]
    p = lax.dot_general(
        pr, ti, (((1,), (1,)), ((), ())), precision=_PREC,
        preferred_element_type=jnp.float32)
    p_ref[0] = p
    tout_ref[0] = t_ref[0, B:, B:] - lax.dot_general(
        p, p, (((1,), (1,)), ((), ())), precision=_PREC,
        preferred_element_type=jnp.float32)


def _solve_logits_body(*refs):
    panel_refs = refs[:NB - 1]
    tinv_refs = refs[NB - 1:2 * NB - 1]
    zf_ref, mean_ref, phi_ref, phij_ref = refs[2 * NB - 1:2 * NB + 3]
    out_ref = refs[2 * NB + 3]

    pvec = {}
    for k in range(NB - 1, -1, -1):
        acc = zf_ref[0, :, k * B:(k + 1) * B]
        for j in range(k + 1, NB):
            rjk = panel_refs[k][0, (j - k - 1) * B:(j - k) * B, :]
            acc = acc - lax.dot_general(
                pvec[j], rjk, (((1,), (0,)), ((), ())), precision=_PREC,
                preferred_element_type=jnp.float32)
        pvec[k] = lax.dot_general(
            acc, tinv_refs[k][0], (((1,), (0,)), ((), ())), precision=_PREC,
            preferred_element_type=jnp.float32)

    logit = jnp.sum(phi_ref[0] * mean_ref[0])
    for k in range(NB):
        logit = logit + jnp.sum(phij_ref[0, :, k * B:(k + 1) * B] * pvec[k])
    pclick = 1.0 / (1.0 + jnp.exp(-logit))
    out_ref[...] = jnp.full((1, 1, B), pclick, jnp.float32)


def _sc_topk_body(p_hbm, out_hbm, p_v, o_v):
    cid = lax.axis_index("c")
    sid = lax.axis_index("s")

    @pl.when(jnp.logical_and(cid == 0, sid == 0))
    def _():
        pltpu.sync_copy(p_hbm, p_v)
        p = p_v[...]
        io = lax.iota(jnp.int32, 16)
        rank = jnp.zeros((16,), jnp.int32)
        for j in range(H):
            pj = jnp.sum(jnp.where(io == j, p, 0.0))
            beats = jnp.logical_or(
                pj > p, jnp.logical_and(pj == p, io > j))
            rank = rank + jnp.where(beats, 1, 0)
        o_v[...] = jnp.where(rank < K, 1.0, 0.0).astype(jnp.float32)
        pltpu.sync_copy(o_v, out_hbm)


def _topk_mask_sc(pclick):
    mesh = plsc.VectorSubcoreMesh(core_axis_name="c", subcore_axis_name="s")
    f = pl.kernel(
        _sc_topk_body, mesh=mesh,
        out_type=jax.ShapeDtypeStruct((H,), jnp.float32),
        scratch_types=[pltpu.VMEM((16,), jnp.float32),
                       pltpu.VMEM((16,), jnp.float32)],
        compiler_params=pltpu.CompilerParams(needs_layout_passes=False))
    return f(pclick)


def _factorize(t):
    panels = []
    tinvs = []
    for k in range(NB):
        m = D - B * k
        r_k, tinv_k = pl.pallas_call(
            _diag_factor_body,
            grid=(1,),
            in_specs=[pl.BlockSpec((H, B, B), lambda i: (0, 0, 0))],
            out_specs=[pl.BlockSpec((H, B, B), lambda i: (0, 0, 0)),
                       pl.BlockSpec((H, B, B), lambda i: (0, 0, 0))],
            out_shape=[jax.ShapeDtypeStruct((H, B, B), jnp.float32),
                       jax.ShapeDtypeStruct((H, B, B), jnp.float32)],
            scratch_shapes=[pltpu.VMEM((H, B, B), jnp.float32)],
        )(t)
        tinvs.append(tinv_k)
        if m > B:
            p_k, t = pl.pallas_call(
                _panel_update_body,
                grid=(H,),
                in_specs=[pl.BlockSpec((1, m, m), lambda h: (h, 0, 0)),
                          pl.BlockSpec((1, B, B), lambda h: (h, 0, 0))],
                out_specs=[
                    pl.BlockSpec((1, m - B, B), lambda h: (h, 0, 0)),
                    pl.BlockSpec((1, m - B, m - B), lambda h: (h, 0, 0)),
                ],
                out_shape=[
                    jax.ShapeDtypeStruct((H, m - B, B), jnp.float32),
                    jax.ShapeDtypeStruct((H, m - B, m - B), jnp.float32),
                ],
            )(t, tinv_k)
            panels.append(p_k)
    return panels, tinvs


def _solve_and_logits(panels, tinvs, zf, mean, phi, phij):
    in_specs = []
    for k in range(NB - 1):
        mk = D - B * (k + 1)
        in_specs.append(pl.BlockSpec((1, mk, B), lambda h: (h, 0, 0)))
    for _ in range(NB):
        in_specs.append(pl.BlockSpec((1, B, B), lambda h: (h, 0, 0)))
    for _ in range(4):
        in_specs.append(pl.BlockSpec((1, 1, D), lambda h: (h, 0, 0)))
    out = pl.pallas_call(
        _solve_logits_body,
        grid=(H,),
        in_specs=in_specs,
        out_specs=pl.BlockSpec((1, 1, B), lambda h: (h, 0, 0)),
        out_shape=jax.ShapeDtypeStruct((H, 1, B), jnp.float32),
    )(*panels, *tinvs, zf, mean, phi, phij)
    return out[:, 0, 0]


def kernel(mean, va, xt, action):
    mean = mean.astype(jnp.float32)
    va = va.astype(jnp.float32)
    xt = xt.astype(jnp.float32)
    action = action.astype(jnp.float32)

    z = jax.random.normal(jax.random.key(42), (H, D), dtype=jnp.float32)
    zf = z[:, ::-1].reshape(H, 1, D)

    phi = jnp.concatenate(
        [jnp.broadcast_to(xt[None, :], (H, DU)), action], axis=1)
    phij = phi[:, ::-1].reshape(H, 1, D)
    phi3 = phi.reshape(H, 1, D)
    mean3 = mean.reshape(H, 1, D)

    t = va[:, ::-1, ::-1]

    panels, tinvs = _factorize(t)
    pclick = _solve_and_logits(panels, tinvs, zf, mean3, phi3, phij)
    return _topk_mask_sc(pclick)


# speedup vs baseline: 12.1466x; 1.2633x over previous
import jax
import jax.numpy as jnp
from jax import lax
from jax.experimental import pallas as pl
from jax.experimental.pallas import tpu as pltpu
from jax.experimental.pallas import tpu_sc as plsc

H = 16
DU = 512
D = 1024
B = 128
NB = D // B
K = 8
_PREC = lax.Precision.HIGHEST


def _bmm(x, y):
    return lax.dot_general(
        x, y, (((2,), (1,)), ((0,), (0,))), precision=_PREC,
        preferred_element_type=jnp.float32)


W = 32
NIB = B // W


def _diag_factor_body(a_ref, tinv_ref, a_s, rt_s):
    a_s[...] = a_ref[...]

    iota_r = lax.broadcasted_iota(jnp.int32, (1, B, 1), 1)
    iota_c = lax.broadcasted_iota(jnp.int32, (1, 1, B), 2)
    iota_w = lax.broadcasted_iota(jnp.int32, (1, W, 1), 1)

    for ib in range(NIB):
        s = ib * W
        rt_s[:, s:s + W, :] = jnp.where(
            iota_c >= s, a_s[:, s:s + W, :], 0.0)

        def inner(j, _, s=s):
            pan = rt_s[:, s:s + W, :]
            g = s + j
            colsel = iota_c == g
            subsel = iota_w == j
            colv = jnp.sum(jnp.where(subsel, pan, 0.0), axis=1, keepdims=True)
            d = jnp.sum(jnp.where(colsel, colv, 0.0), axis=2, keepdims=True)
            rs = lax.rsqrt(d)
            lrow = jnp.where(iota_c >= g, colv * rs, 0.0)
            lsubraw = jnp.sum(jnp.where(colsel, pan, 0.0), axis=2,
                              keepdims=True)
            lsub = jnp.where(iota_w > j, lsubraw * rs, 0.0)
            upd = pan - lsub * lrow
            rt_s[:, s:s + W, :] = jnp.where(subsel, lrow, upd)
            return 0

        lax.fori_loop(0, W, inner, 0)

        if s + W < B:
            panf = rt_s[:, s:s + W, :]
            a_s[:, :, s + W:] = a_s[:, :, s + W:] - lax.dot_general(
                panf, panf[:, :, s + W:], (((1,), (1,)), ((0,), (0,))),
                precision=_PREC, preferred_element_type=jnp.float32)

    rt = rt_s[...]
    eye = jnp.where(iota_r == iota_c, 1.0, 0.0)
    diag = jnp.sum(jnp.where(iota_r == iota_c, rt, 0.0), axis=2)
    n = rt / diag[:, None, :] - eye
    p = eye - n
    m = n
    for _ in range(6):
        m = _bmm(m, m)
        p = _bmm(p, eye + m)
    tinv_ref[...] = p / diag[:, :, None]


def _panel_update_body(t_ref, tinv_ref, p_ref, tout_ref):
    pr = t_ref[0, B:, :B]
    ti = tinv_ref[0]
    p = lax.dot_general(
        pr, ti, (((1,), (0,)), ((), ())), precision=_PREC,
        preferred_element_type=jnp.float32)
    p_ref[0] = p
    tout_ref[0] = t_ref[0, B:, B:] - lax.dot_general(
        p, p, (((1,), (1,)), ((), ())), precision=_PREC,
        preferred_element_type=jnp.float32)


def _solve_logits_body(*refs):
    panel_refs = refs[:NB - 1]
    tinv_refs = refs[NB - 1:2 * NB - 1]
    zf_ref, mean_ref, phi_ref, phij_ref = refs[2 * NB - 1:2 * NB + 3]
    out_ref = refs[2 * NB + 3]

    pvec = {}
    for k in range(NB - 1, -1, -1):
        acc = zf_ref[0, :, k * B:(k + 1) * B]
        for j in range(k + 1, NB):
            rjk = panel_refs[k][0, (j - k - 1) * B:(j - k) * B, :]
            acc = acc - lax.dot_general(
                pvec[j], rjk, (((1,), (0,)), ((), ())), precision=_PREC,
                preferred_element_type=jnp.float32)
        pvec[k] = lax.dot_general(
            acc, tinv_refs[k][0], (((1,), (1,)), ((), ())), precision=_PREC,
            preferred_element_type=jnp.float32)

    logit = jnp.sum(phi_ref[0] * mean_ref[0])
    for k in range(NB):
        logit = logit + jnp.sum(phij_ref[0, :, k * B:(k + 1) * B] * pvec[k])
    pclick = 1.0 / (1.0 + jnp.exp(-logit))
    out_ref[...] = jnp.full((1, 1, B), pclick, jnp.float32)


def _sc_topk_body(p_hbm, out_hbm, p_v, o_v):
    cid = lax.axis_index("c")
    sid = lax.axis_index("s")

    @pl.when(jnp.logical_and(cid == 0, sid == 0))
    def _():
        pltpu.sync_copy(p_hbm, p_v)
        p = p_v[...]
        io = lax.iota(jnp.int32, 16)
        rank = jnp.zeros((16,), jnp.int32)
        for j in range(H):
            pj = jnp.sum(jnp.where(io == j, p, 0.0))
            beats = jnp.logical_or(
                pj > p, jnp.logical_and(pj == p, io > j))
            rank = rank + jnp.where(beats, 1, 0)
        o_v[...] = jnp.where(rank < K, 1.0, 0.0).astype(jnp.float32)
        pltpu.sync_copy(o_v, out_hbm)


def _topk_mask_sc(pclick):
    mesh = plsc.VectorSubcoreMesh(core_axis_name="c", subcore_axis_name="s")
    f = pl.kernel(
        _sc_topk_body, mesh=mesh,
        out_type=jax.ShapeDtypeStruct((H,), jnp.float32),
        scratch_types=[pltpu.VMEM((16,), jnp.float32),
                       pltpu.VMEM((16,), jnp.float32)],
        compiler_params=pltpu.CompilerParams(needs_layout_passes=False))
    return f(pclick)


def _factorize(t):
    panels = []
    tinvs = []
    for k in range(NB):
        m = D - B * k
        tinv_k = pl.pallas_call(
            _diag_factor_body,
            grid=(1,),
            in_specs=[pl.BlockSpec((H, B, B), lambda i: (0, 0, 0))],
            out_specs=pl.BlockSpec((H, B, B), lambda i: (0, 0, 0)),
            out_shape=jax.ShapeDtypeStruct((H, B, B), jnp.float32),
            scratch_shapes=[pltpu.VMEM((H, B, B), jnp.float32),
                            pltpu.VMEM((H, B, B), jnp.float32)],
        )(t)
        tinvs.append(tinv_k)
        if m > B:
            p_k, t = pl.pallas_call(
                _panel_update_body,
                grid=(H,),
                in_specs=[pl.BlockSpec((1, m, m), lambda h: (h, 0, 0)),
                          pl.BlockSpec((1, B, B), lambda h: (h, 0, 0))],
                out_specs=[
                    pl.BlockSpec((1, m - B, B), lambda h: (h, 0, 0)),
                    pl.BlockSpec((1, m - B, m - B), lambda h: (h, 0, 0)),
                ],
                out_shape=[
                    jax.ShapeDtypeStruct((H, m - B, B), jnp.float32),
                    jax.ShapeDtypeStruct((H, m - B, m - B), jnp.float32),
                ],
            )(t, tinv_k)
            panels.append(p_k)
    return panels, tinvs


def _solve_and_logits(panels, tinvs, zf, mean, phi, phij):
    in_specs = []
    for k in range(NB - 1):
        mk = D - B * (k + 1)
        in_specs.append(pl.BlockSpec((1, mk, B), lambda h: (h, 0, 0)))
    for _ in range(NB):
        in_specs.append(pl.BlockSpec((1, B, B), lambda h: (h, 0, 0)))
    for _ in range(4):
        in_specs.append(pl.BlockSpec((1, 1, D), lambda h: (h, 0, 0)))
    out = pl.pallas_call(
        _solve_logits_body,
        grid=(H,),
        in_specs=in_specs,
        out_specs=pl.BlockSpec((1, 1, B), lambda h: (h, 0, 0)),
        out_shape=jax.ShapeDtypeStruct((H, 1, B), jnp.float32),
    )(*panels, *tinvs, zf, mean, phi, phij)
    return out[:, 0, 0]


def kernel(mean, va, xt, action):
    mean = mean.astype(jnp.float32)
    va = va.astype(jnp.float32)
    xt = xt.astype(jnp.float32)
    action = action.astype(jnp.float32)

    z = jax.random.normal(jax.random.key(42), (H, D), dtype=jnp.float32)
    zf = z[:, ::-1].reshape(H, 1, D)

    phi = jnp.concatenate(
        [jnp.broadcast_to(xt[None, :], (H, DU)), action], axis=1)
    phij = phi[:, ::-1].reshape(H, 1, D)
    phi3 = phi.reshape(H, 1, D)
    mean3 = mean.reshape(H, 1, D)

    t = va[:, ::-1, ::-1]

    panels, tinvs = _factorize(t)
    pclick = _solve_and_logits(panels, tinvs, zf, mean3, phi3, phij)
    return _topk_mask_sc(pclick)


# speedup vs baseline: 22.6306x; 1.8631x over previous
import jax
import jax.numpy as jnp
from jax import lax
from jax.experimental import pallas as pl
from jax.experimental.pallas import tpu as pltpu
from jax.experimental.pallas import tpu_sc as plsc

H = 16
DU = 512
D = 1024
B = 128
NB = D // B
K = 8
_PREC = lax.Precision.HIGHEST


def _bmm(x, y):
    return lax.dot_general(
        x, y, (((2,), (1,)), ((0,), (0,))), precision=_PREC,
        preferred_element_type=jnp.float32)


W = 32
NIB = B // W


def _diag_factor_body(a_ref, tinv_ref, a_s, rt_s):
    a_s[...] = a_ref[...]

    iota_r = lax.broadcasted_iota(jnp.int32, (1, B, 1), 1)
    iota_c = lax.broadcasted_iota(jnp.int32, (1, 1, B), 2)
    iota_w = lax.broadcasted_iota(jnp.int32, (1, W, 1), 1)

    for ib in range(NIB):
        sd = B - (ib + 1) * W
        rt_s[:, sd:sd + W, :] = jnp.where(
            iota_c < sd + W, a_s[:, sd:sd + W, :], 0.0)

        def inner(jj, _, sd=sd):
            pan = rt_s[:, sd:sd + W, :]
            j = W - 1 - jj
            g = sd + j
            colsel = iota_c == g
            subsel = iota_w == j
            colv = jnp.sum(jnp.where(subsel, pan, 0.0), axis=1, keepdims=True)
            d = jnp.sum(jnp.where(colsel, colv, 0.0), axis=2, keepdims=True)
            rs = lax.rsqrt(d)
            urow = jnp.where(iota_c <= g, colv * rs, 0.0)
            usubraw = jnp.sum(jnp.where(colsel, pan, 0.0), axis=2,
                              keepdims=True)
            usub = jnp.where(iota_w < j, usubraw * rs, 0.0)
            upd = pan - usub * urow
            rt_s[:, sd:sd + W, :] = jnp.where(subsel, urow, upd)
            return 0

        lax.fori_loop(0, W, inner, 0)

        if sd > 0:
            panf = rt_s[:, sd:sd + W, :]
            a_s[:, :, :sd] = a_s[:, :, :sd] - lax.dot_general(
                panf, panf[:, :, :sd], (((1,), (1,)), ((0,), (0,))),
                precision=_PREC, preferred_element_type=jnp.float32)

    rt = rt_s[...]
    eye = jnp.where(iota_r == iota_c, 1.0, 0.0)
    diag = jnp.sum(jnp.where(iota_r == iota_c, rt, 0.0), axis=2)
    n = rt / diag[:, :, None] - eye
    p = eye - n
    m = n
    for _ in range(6):
        m = _bmm(m, m)
        p = _bmm(p, eye + m)
    tinv_ref[...] = p / diag[:, None, :]


def _panel_update_body(t_ref, tinv_ref, p_ref, tout_ref):
    mm = t_ref.shape[1]
    pr = t_ref[0, :mm - B, mm - B:]
    g = tinv_ref[0]
    p = lax.dot_general(
        pr, g, (((1,), (0,)), ((), ())), precision=_PREC,
        preferred_element_type=jnp.float32)
    p_ref[0] = p
    tout_ref[0] = t_ref[0, :mm - B, :mm - B] - lax.dot_general(
        p, p, (((1,), (1,)), ((), ())), precision=_PREC,
        preferred_element_type=jnp.float32)


def _solve_logits_body(*refs):
    panel_refs = refs[:NB - 1]
    tinv_refs = refs[NB - 1:2 * NB - 1]
    z_ref, mean_ref, phi_ref = refs[2 * NB - 1:2 * NB + 2]
    out_ref = refs[2 * NB + 2]

    y = {}
    for k in range(NB):
        acc = z_ref[0, :, k * B:(k + 1) * B]
        for j in range(k):
            ujk = panel_refs[k - 1][0, j * B:(j + 1) * B, :]
            acc = acc - lax.dot_general(
                y[j], ujk, (((1,), (0,)), ((), ())), precision=_PREC,
                preferred_element_type=jnp.float32)
        y[k] = lax.dot_general(
            acc, tinv_refs[k][0], (((1,), (1,)), ((), ())), precision=_PREC,
            preferred_element_type=jnp.float32)

    logit = jnp.sum(phi_ref[0] * mean_ref[0])
    for k in range(NB):
        logit = logit + jnp.sum(phi_ref[0, :, k * B:(k + 1) * B] * y[k])
    pclick = 1.0 / (1.0 + jnp.exp(-logit))
    out_ref[...] = jnp.full((1, 1, B), pclick, jnp.float32)


def _sc_topk_body(p_hbm, out_hbm, p_v, o_v):
    cid = lax.axis_index("c")
    sid = lax.axis_index("s")

    @pl.when(jnp.logical_and(cid == 0, sid == 0))
    def _():
        pltpu.sync_copy(p_hbm, p_v)
        p = p_v[...]
        io = lax.iota(jnp.int32, 16)
        rank = jnp.zeros((16,), jnp.int32)
        for j in range(H):
            pj = jnp.sum(jnp.where(io == j, p, 0.0))
            beats = jnp.logical_or(
                pj > p, jnp.logical_and(pj == p, io > j))
            rank = rank + jnp.where(beats, 1, 0)
        o_v[...] = jnp.where(rank < K, 1.0, 0.0).astype(jnp.float32)
        pltpu.sync_copy(o_v, out_hbm)


def _topk_mask_sc(pclick):
    mesh = plsc.VectorSubcoreMesh(core_axis_name="c", subcore_axis_name="s")
    f = pl.kernel(
        _sc_topk_body, mesh=mesh,
        out_type=jax.ShapeDtypeStruct((H,), jnp.float32),
        scratch_types=[pltpu.VMEM((16,), jnp.float32),
                       pltpu.VMEM((16,), jnp.float32)],
        compiler_params=pltpu.CompilerParams(needs_layout_passes=False))
    return f(pclick)


def _factorize(t):
    panels = [None] * NB
    tinvs = [None] * NB
    for k in range(NB - 1, -1, -1):
        m = (k + 1) * B
        tinvs[k] = pl.pallas_call(
            _diag_factor_body,
            grid=(1,),
            in_specs=[pl.BlockSpec((H, B, B), lambda i, k=k: (0, k, k))],
            out_specs=pl.BlockSpec((H, B, B), lambda i: (0, 0, 0)),
            out_shape=jax.ShapeDtypeStruct((H, B, B), jnp.float32),
            scratch_shapes=[pltpu.VMEM((H, B, B), jnp.float32),
                            pltpu.VMEM((H, B, B), jnp.float32)],
        )(t)
        if k > 0:
            panels[k], t = pl.pallas_call(
                _panel_update_body,
                grid=(H,),
                in_specs=[pl.BlockSpec((1, m, m), lambda h: (h, 0, 0)),
                          pl.BlockSpec((1, B, B), lambda h: (h, 0, 0))],
                out_specs=[
                    pl.BlockSpec((1, m - B, B), lambda h: (h, 0, 0)),
                    pl.BlockSpec((1, m - B, m - B), lambda h: (h, 0, 0)),
                ],
                out_shape=[
                    jax.ShapeDtypeStruct((H, m - B, B), jnp.float32),
                    jax.ShapeDtypeStruct((H, m - B, m - B), jnp.float32),
                ],
            )(t, tinvs[k])
    return panels, tinvs


def _solve_and_logits(panels, tinvs, z3, mean, phi):
    in_specs = []
    for k in range(1, NB):
        in_specs.append(
            pl.BlockSpec((1, k * B, B), lambda h: (h, 0, 0)))
    for _ in range(NB):
        in_specs.append(pl.BlockSpec((1, B, B), lambda h: (h, 0, 0)))
    for _ in range(3):
        in_specs.append(pl.BlockSpec((1, 1, D), lambda h: (h, 0, 0)))
    out = pl.pallas_call(
        _solve_logits_body,
        grid=(H,),
        in_specs=in_specs,
        out_specs=pl.BlockSpec((1, 1, B), lambda h: (h, 0, 0)),
        out_shape=jax.ShapeDtypeStruct((H, 1, B), jnp.float32),
    )(*panels[1:], *tinvs, z3, mean, phi)
    return out[:, 0, 0]


def kernel(mean, va, xt, action):
    mean = mean.astype(jnp.float32)
    va = va.astype(jnp.float32)
    xt = xt.astype(jnp.float32)
    action = action.astype(jnp.float32)

    z = jax.random.normal(jax.random.key(42), (H, D), dtype=jnp.float32)
    z3 = z.reshape(H, 1, D)

    phi = jnp.concatenate(
        [jnp.broadcast_to(xt[None, :], (H, DU)), action], axis=1)
    phi3 = phi.reshape(H, 1, D)
    mean3 = mean.reshape(H, 1, D)

    panels, tinvs = _factorize(va)
    pclick = _solve_and_logits(panels, tinvs, z3, mean3, phi3)
    return _topk_mask_sc(pclick)
